# Initial kernel scaffold; baseline (speedup 1.0000x reference)
import jax
import jax.numpy as jnp
from jax.experimental import pallas as pl


def kernel(inputs, W):
    raise NotImplementedError("write your pallas kernel here")


# speedup vs baseline: 5.7075x; 5.7075x over previous
import jax
import jax.numpy as jnp
from jax.experimental import pallas as pl
from jax.experimental.pallas import tpu as pltpu

NUM_E = 8192
DIM = 256
BATCH = 4096
CCOST = 0.25
BB = 256
NB = BATCH // BB
_PREC = jax.lax.Precision.DEFAULT


def _vq_body(x_ref, w_ref, enc_ref, qst_ref, loss_ref, perp_ref,
             acc_ref, counts_ref):
    i = pl.program_id(0)
    x = x_ref[...]
    w = w_ref[...]
    xsq = jnp.sum(x * x, axis=1, keepdims=True)
    wsq = jnp.sum(w * w, axis=1)
    m = jax.lax.dot_general(x, w, (((1,), (1,)), ((), ())),
                            precision=_PREC,
                            preferred_element_type=jnp.float32)
    d = (xsq + wsq[None, :]) - 2.0 * m
    dmin = jnp.min(d, axis=1, keepdims=True)
    eio = jax.lax.broadcasted_iota(jnp.int32, (BB, NUM_E), 1)
    idx = jnp.min(jnp.where(d == dmin, eio, NUM_E), axis=1, keepdims=True)
    enc = (eio == idx).astype(jnp.float32)
    enc_ref[...] = enc
    q = jax.lax.dot_general(enc, w, (((1,), (0,)), ((), ())),
                            precision=_PREC,
                            preferred_element_type=jnp.float32)
    qst_ref[...] = x + (q - x)
    diff = q - x

    @pl.when(i == 0)
    def _init():
        acc_ref[...] = jnp.zeros_like(acc_ref)
        counts_ref[...] = jnp.zeros_like(counts_ref)

    acc_ref[...] += jnp.sum(diff * diff, axis=(0, 1), keepdims=True)
    counts_ref[...] += jnp.sum(enc, axis=0, keepdims=True)

    @pl.when(i == NB - 1)
    def _fin():
        mean_sq = acc_ref[...] / float(BATCH * DIM)
        loss_ref[...] = mean_sq + CCOST * mean_sq
        p = counts_ref[...] / float(BATCH)
        ent = jnp.sum(p * jnp.log(p + 1e-10), axis=1, keepdims=True)
        perp_ref[...] = jnp.exp(-ent)


def kernel(inputs, W):
    x = inputs.reshape(BATCH, DIM)
    enc, qst, loss, perp = pl.pallas_call(
        _vq_body,
        grid=(NB,),
        in_specs=[
            pl.BlockSpec((BB, DIM), lambda i: (i, 0)),
            pl.BlockSpec((NUM_E, DIM), lambda i: (0, 0)),
        ],
        out_specs=[
            pl.BlockSpec((BB, NUM_E), lambda i: (i, 0)),
            pl.BlockSpec((BB, DIM), lambda i: (i, 0)),
            pl.BlockSpec((1, 1), lambda i: (0, 0)),
            pl.BlockSpec((1, 1), lambda i: (0, 0)),
        ],
        out_shape=[
            jax.ShapeDtypeStruct((BATCH, NUM_E), jnp.float32),
            jax.ShapeDtypeStruct((BATCH, DIM), jnp.float32),
            jax.ShapeDtypeStruct((1, 1), jnp.float32),
            jax.ShapeDtypeStruct((1, 1), jnp.float32),
        ],
        scratch_shapes=[
            pltpu.VMEM((1, 1), jnp.float32),
            pltpu.VMEM((1, NUM_E), jnp.float32),
        ],
    )(x, W)
    return (loss[0, 0], qst.reshape(inputs.shape), perp[0, 0], enc)


# speedup vs baseline: 6.5883x; 1.1543x over previous
import functools

import jax
import jax.numpy as jnp
from jax.experimental import pallas as pl
from jax.experimental.pallas import tpu as pltpu
from jax.experimental.pallas import tpu_sc as plsc

NUM_E = 8192
DIM = 256
BATCH = 4096
CCOST = 0.25
BB = 256
NB = BATCH // BB
_PREC = jax.lax.Precision.DEFAULT

_SC = plsc.get_sparse_core_info()
_NW = _SC.num_cores * _SC.num_subcores
_BPW = BATCH // _NW


def _vq_body(x_ref, w_ref, enc_ref, idx_ref, perp_ref,
             wsq_ref, eio_ref, counts_ref):
    i = pl.program_id(0)

    @pl.when(i == 0)
    def _prologue():
        w0 = w_ref[...]
        wsq_ref[...] = jnp.sum(w0 * w0, axis=1)[None, :]
        eio_ref[...] = jax.lax.broadcasted_iota(jnp.int32, (1, NUM_E), 1)
        counts_ref[...] = jnp.zeros_like(counts_ref)

    x = x_ref[...]
    w = w_ref[...]
    xsq = jnp.sum(x * x, axis=1, keepdims=True)
    m = jax.lax.dot_general(x, w, (((1,), (1,)), ((), ())),
                            precision=_PREC,
                            preferred_element_type=jnp.float32)
    d = (xsq + wsq_ref[...]) - 2.0 * m
    dmin = jnp.min(d, axis=1, keepdims=True)
    eio = eio_ref[...]
    idx = jnp.min(jnp.where(d == dmin, eio, NUM_E), axis=1, keepdims=True)
    enc = (eio == idx).astype(jnp.float32)
    enc_ref[...] = enc
    idx_ref[...] = idx
    counts_ref[...] += jnp.sum(enc, axis=0, keepdims=True)

    @pl.when(i == NB - 1)
    def _fin():
        p = counts_ref[...] / float(BATCH)
        ent = jnp.sum(p * jnp.log(p + 1e-10), axis=1, keepdims=True)
        perp_ref[...] = jnp.exp(-ent)


def _gather_body(w_hbm, idx_hbm, q_hbm, idx_v, rows_v, sem):
    wid = jax.lax.axis_index("s") * _SC.num_cores + jax.lax.axis_index("c")
    base = wid * _BPW
    pltpu.sync_copy(idx_hbm.at[pl.ds(base, _BPW)], idx_v)
    pltpu.async_copy(w_hbm.at[idx_v], rows_v, sem).wait()
    pltpu.sync_copy(rows_v, q_hbm.at[pl.ds(base, _BPW)])


def _epi_body(x_ref, q_ref, qst_ref, loss_ref):
    x = x_ref[...]
    q = q_ref[...].astype(jnp.bfloat16).astype(jnp.float32)
    qst_ref[...] = x + (q - x)
    diff = q - x
    s = jnp.sum(diff * diff, axis=(0, 1), keepdims=True)
    mean_sq = s / float(BATCH * DIM)
    loss_ref[...] = mean_sq + CCOST * mean_sq


def kernel(inputs, W):
    x = inputs.reshape(BATCH, DIM)
    enc, idx, perp = pl.pallas_call(
        _vq_body,
        grid=(NB,),
        in_specs=[
            pl.BlockSpec((BB, DIM), lambda i: (i, 0)),
            pl.BlockSpec((NUM_E, DIM), lambda i: (0, 0)),
        ],
        out_specs=[
            pl.BlockSpec((BB, NUM_E), lambda i: (i, 0)),
            pl.BlockSpec((BB, 1), lambda i: (i, 0)),
            pl.BlockSpec((1, 1), lambda i: (0, 0)),
        ],
        out_shape=[
            jax.ShapeDtypeStruct((BATCH, NUM_E), jnp.float32),
            jax.ShapeDtypeStruct((BATCH, 1), jnp.int32),
            jax.ShapeDtypeStruct((1, 1), jnp.float32),
        ],
        scratch_shapes=[
            pltpu.VMEM((1, NUM_E), jnp.float32),
            pltpu.VMEM((1, NUM_E), jnp.int32),
            pltpu.VMEM((1, NUM_E), jnp.float32),
        ],
    )(x, W)

    mesh = plsc.VectorSubcoreMesh(core_axis_name="c", subcore_axis_name="s")
    q = pl.kernel(
        _gather_body,
        out_type=jax.ShapeDtypeStruct((BATCH, DIM), jnp.float32),
        mesh=mesh,
        scratch_types=[
            pltpu.VMEM((_BPW,), jnp.int32),
            pltpu.VMEM((_BPW, DIM), jnp.float32),
            pltpu.SemaphoreType.DMA,
        ],
    )(W, idx.reshape(BATCH))

    qst, loss = pl.pallas_call(
        _epi_body,
        out_shape=[
            jax.ShapeDtypeStruct((BATCH, DIM), jnp.float32),
            jax.ShapeDtypeStruct((1, 1), jnp.float32),
        ],
    )(x, q)
    return (loss[0, 0], qst.reshape(inputs.shape), perp[0, 0], enc)
